# baseline (speedup 1.0000x reference)
import functools

import jax
import jax.numpy as jnp
from jax import lax
from jax.experimental import pallas as pl
from jax.experimental.pallas import tpu as pltpu
from jax.experimental.pallas import tpu_sc as plsc

_D = 128
_NW = 32
_CHUNK = 128
_NCHUNK = 25
_PER_W = _CHUNK * _NCHUNK
_B_PAD = _PER_W * _NW

_mesh = plsc.VectorSubcoreMesh(core_axis_name="c", subcore_axis_name="s")


@functools.partial(
    pl.kernel,
    out_type=jax.ShapeDtypeStruct((_B_PAD, _D), jnp.float32),
    mesh=_mesh,
    scratch_types=[
        pltpu.VMEM((_NCHUNK, _CHUNK), jnp.int32),
        pltpu.VMEM((_CHUNK, _D), jnp.float32),
        pltpu.SemaphoreType.DMA,
    ],
)
def _sc_gather(x_hbm, idx_hbm, out_hbm, idx_v, buf, sem):
    wid = lax.axis_index("s") * 2 + lax.axis_index("c")
    base = wid * _PER_W
    pltpu.sync_copy(idx_hbm.at[wid], idx_v)

    def body(c, carry):
        pltpu.async_copy(x_hbm.at[idx_v.at[c]], buf, sem).wait()
        pltpu.sync_copy(buf, out_hbm.at[pl.ds(base + c * _CHUNK, _CHUNK)])
        return carry

    lax.fori_loop(0, _NCHUNK, body, 0)


def kernel(x, upsamples):
    n = upsamples.shape[0]
    idx = upsamples[:, 0].astype(jnp.int32)
    idx = jnp.concatenate([idx, jnp.zeros((_B_PAD - n,), jnp.int32)])
    idx = idx.reshape(_NW, _NCHUNK, _CHUNK)
    out = _sc_gather(x, idx)
    return out[:n]


# speedup vs baseline: 1.0829x; 1.0829x over previous
import functools

import jax
import jax.numpy as jnp
from jax import lax
from jax.experimental import pallas as pl
from jax.experimental.pallas import tpu as pltpu
from jax.experimental.pallas import tpu_sc as plsc

_D = 128
_NW = 32
_CHUNK = 128
_NBUF = 5
_NGRP = 5
_NCHUNK = _NBUF * _NGRP
_PER_W = _CHUNK * _NCHUNK
_B_PAD = _PER_W * _NW

_mesh = plsc.VectorSubcoreMesh(core_axis_name="c", subcore_axis_name="s")


@functools.partial(
    pl.kernel,
    out_type=jax.ShapeDtypeStruct((_B_PAD, _D), jnp.float32),
    mesh=_mesh,
    scratch_types=[
        pltpu.VMEM((_NCHUNK, _CHUNK), jnp.int32),
        [pltpu.VMEM((_CHUNK, _D), jnp.float32) for _ in range(_NBUF)],
        [pltpu.SemaphoreType.DMA for _ in range(_NBUF)],
        [pltpu.SemaphoreType.DMA for _ in range(_NBUF)],
    ],
)
def _sc_gather(x_hbm, idx_hbm, out_hbm, idx_v, bufs, gsems, ssems):
    wid = lax.axis_index("s") * 2 + lax.axis_index("c")
    base = wid * _PER_W
    pltpu.sync_copy(idx_hbm.at[wid], idx_v)

    def fire_gather(c, b):
        pltpu.async_copy(x_hbm.at[idx_v.at[c]], bufs[b], gsems[b])

    def fire_scatter(c, b):
        pltpu.async_copy(
            bufs[b], out_hbm.at[pl.ds(base + c * _CHUNK, _CHUNK)], ssems[b]
        )

    def drain(sem, b):
        pltpu.make_async_copy(x_hbm.at[pl.ds(0, _CHUNK)], bufs[b], sem).wait()

    for b in range(_NBUF):
        fire_gather(b, b)

    def body(i, carry):
        for b in range(_NBUF):
            drain(gsems[b], b)
            fire_scatter(i * _NBUF + b, b)
        for b in range(_NBUF):
            drain(ssems[b], b)
            fire_gather((i + 1) * _NBUF + b, b)
        return carry

    lax.fori_loop(0, _NGRP - 1, body, 0)

    for b in range(_NBUF):
        drain(gsems[b], b)
        fire_scatter((_NGRP - 1) * _NBUF + b, b)
    for b in range(_NBUF):
        drain(ssems[b], b)


def kernel(x, upsamples):
    n = upsamples.shape[0]
    idx = upsamples[:, 0].astype(jnp.int32)
    idx = jnp.concatenate([idx, jnp.zeros((_B_PAD - n,), jnp.int32)])
    idx = idx.reshape(_NW, _NCHUNK, _CHUNK)
    out = _sc_gather(x, idx)
    return out[:n]


# speedup vs baseline: 3.2750x; 3.0242x over previous
import functools

import jax
import jax.numpy as jnp
from jax import lax
from jax.experimental import pallas as pl
from jax.experimental.pallas import tpu as pltpu
from jax.experimental.pallas import tpu_sc as plsc

_D = 128
_NW = 32
_CHUNK = 128
_NBUF = 5
_NGRP = 5
_NCHUNK = _NBUF * _NGRP
_PER_W = _CHUNK * _NCHUNK
_B = 100000
_B_PAD = _PER_W * _NW
_LAST_FULL = (_B - 31 * _PER_W) // _CHUNK
_LAST_REM = _B - 31 * _PER_W - _LAST_FULL * _CHUNK

_mesh = plsc.VectorSubcoreMesh(core_axis_name="c", subcore_axis_name="s")


@functools.partial(
    pl.kernel,
    out_type=jax.ShapeDtypeStruct((_B, _D), jnp.float32),
    mesh=_mesh,
    scratch_types=[
        pltpu.VMEM((_NCHUNK, _CHUNK), jnp.int32),
        [pltpu.VMEM((_CHUNK, _D), jnp.float32) for _ in range(_NBUF)],
        [pltpu.SemaphoreType.DMA for _ in range(_NBUF)],
        [pltpu.SemaphoreType.DMA for _ in range(_NBUF)],
    ],
)
def _sc_gather(x_hbm, idx_hbm, out_hbm, idx_v, bufs, gsems, ssems):
    wid = lax.axis_index("s") * 2 + lax.axis_index("c")
    base = wid * _PER_W
    pltpu.sync_copy(idx_hbm.at[wid], idx_v)

    def fire_gather(c, b):
        pltpu.async_copy(x_hbm.at[idx_v.at[c]], bufs[b], gsems[b])

    def fire_scatter(c, b):
        pltpu.async_copy(
            bufs[b], out_hbm.at[pl.ds(base + c * _CHUNK, _CHUNK)], ssems[b]
        )

    def drain(sem, b):
        pltpu.make_async_copy(x_hbm.at[pl.ds(0, _CHUNK)], bufs[b], sem).wait()

    @pl.when(wid < _NW - 1)
    def _full_worker():
        for b in range(_NBUF):
            fire_gather(b, b)

        def body(i, carry):
            for b in range(_NBUF):
                drain(gsems[b], b)
                fire_scatter(i * _NBUF + b, b)
            for b in range(_NBUF):
                drain(ssems[b], b)
                fire_gather((i + 1) * _NBUF + b, b)
            return carry

        lax.fori_loop(0, _NGRP - 1, body, 0)

        for b in range(_NBUF):
            drain(gsems[b], b)
            fire_scatter((_NGRP - 1) * _NBUF + b, b)
        for b in range(_NBUF):
            drain(ssems[b], b)

    @pl.when(wid == _NW - 1)
    def _tail_worker():
        def body(c, carry):
            pltpu.async_copy(x_hbm.at[idx_v.at[c]], bufs[0], gsems[0]).wait()
            pltpu.sync_copy(bufs[0], out_hbm.at[pl.ds(base + c * _CHUNK, _CHUNK)])
            return carry

        lax.fori_loop(0, _LAST_FULL, body, 0)
        pltpu.async_copy(x_hbm.at[idx_v.at[_LAST_FULL]], bufs[0], gsems[0]).wait()
        pltpu.sync_copy(
            bufs[0].at[pl.ds(0, _LAST_REM)],
            out_hbm.at[pl.ds(base + _LAST_FULL * _CHUNK, _LAST_REM)],
        )


def kernel(x, upsamples):
    n = upsamples.shape[0]
    idx = upsamples[:, 0].astype(jnp.int32)
    idx = jnp.concatenate([idx, jnp.zeros((_B_PAD - n,), jnp.int32)])
    idx = idx.reshape(_NW, _NCHUNK, _CHUNK)
    x_src = jnp.concatenate([x, jnp.zeros((8, _D), x.dtype)])
    out = _sc_gather(x_src, idx)
    return out


# speedup vs baseline: 4.1974x; 1.2816x over previous
import functools

import jax
import jax.numpy as jnp
from jax import lax
from jax.experimental import pallas as pl
from jax.experimental.pallas import tpu as pltpu
from jax.experimental.pallas import tpu_sc as plsc

_D = 128
_NW = 32
_CHUNK = 128
_NBUF = 5
_NGRP = 5
_NCHUNK = _NBUF * _NGRP
_PER_W = _CHUNK * _NCHUNK
_B = 100000
_B_PAD = _PER_W * _NW
_LAST_FULL = (_B - 31 * _PER_W) // _CHUNK
_LAST_REM = _B - 31 * _PER_W - _LAST_FULL * _CHUNK

_mesh = plsc.VectorSubcoreMesh(core_axis_name="c", subcore_axis_name="s")


@functools.partial(
    pl.kernel,
    out_type=jax.ShapeDtypeStruct((_B, _D), jnp.float32),
    mesh=_mesh,
    scratch_types=[
        pltpu.VMEM((_NCHUNK, _CHUNK), jnp.int32),
        [pltpu.VMEM((_CHUNK, _D), jnp.float32) for _ in range(_NBUF)],
        [pltpu.SemaphoreType.DMA for _ in range(_NBUF)],
        [pltpu.SemaphoreType.DMA for _ in range(_NBUF)],
    ],
)
def _sc_gather(x_hbm, idx_hbm, out_hbm, idx_v, bufs, gsems, ssems):
    wid = lax.axis_index("s") * 2 + lax.axis_index("c")
    base = wid * _PER_W
    pltpu.sync_copy(idx_hbm.at[wid], idx_v)

    def fire_gather(c, b):
        pltpu.async_copy(x_hbm.at[idx_v.at[c]], bufs[b], gsems[b])

    def fire_scatter(c, b):
        pltpu.async_copy(
            bufs[b], out_hbm.at[pl.ds(base + c * _CHUNK, _CHUNK)], ssems[b]
        )

    def drain(sem, b):
        pltpu.make_async_copy(x_hbm.at[pl.ds(0, _CHUNK)], bufs[b], sem).wait()

    @pl.when(wid < _NW - 1)
    def _full_worker():
        for b in range(_NBUF):
            fire_gather(b, b)

        def body(i, carry):
            for b in range(_NBUF):
                drain(gsems[b], b)
                fire_scatter(i * _NBUF + b, b)
            for b in range(_NBUF):
                drain(ssems[b], b)
                fire_gather((i + 1) * _NBUF + b, b)
            return carry

        lax.fori_loop(0, _NGRP - 1, body, 0)

        for b in range(_NBUF):
            drain(gsems[b], b)
            fire_scatter((_NGRP - 1) * _NBUF + b, b)
        for b in range(_NBUF):
            drain(ssems[b], b)

    @pl.when(wid == _NW - 1)
    def _tail_worker():
        def body(c, carry):
            pltpu.async_copy(x_hbm.at[idx_v.at[c]], bufs[0], gsems[0]).wait()
            pltpu.sync_copy(bufs[0], out_hbm.at[pl.ds(base + c * _CHUNK, _CHUNK)])
            return carry

        lax.fori_loop(0, _LAST_FULL, body, 0)
        pltpu.async_copy(x_hbm.at[idx_v.at[_LAST_FULL]], bufs[0], gsems[0]).wait()
        pltpu.sync_copy(
            bufs[0].at[pl.ds(0, _LAST_REM)],
            out_hbm.at[pl.ds(base + _LAST_FULL * _CHUNK, _LAST_REM)],
        )


def kernel(x, upsamples):
    n = upsamples.shape[0]
    idx = upsamples[:, 0].astype(jnp.int32)
    idx = jnp.concatenate([idx, jnp.zeros((_B_PAD - n,), jnp.int32)])
    idx = idx.reshape(_NW, _NCHUNK, _CHUNK)
    out = _sc_gather(x, idx)
    return out


# speedup vs baseline: 4.4857x; 1.0687x over previous
import functools

import jax
import jax.numpy as jnp
from jax import lax
from jax.experimental import pallas as pl
from jax.experimental.pallas import tpu as pltpu
from jax.experimental.pallas import tpu_sc as plsc

_D = 128
_NW = 32
_CHUNK = 64
_NBUF = 10
_NGRP = 5
_NCHUNK = _NBUF * _NGRP
_PER_W = _CHUNK * _NCHUNK
_B = 100000
_B_PAD = _PER_W * _NW
_LAST_FULL = (_B - 31 * _PER_W) // _CHUNK
_LAST_REM = _B - 31 * _PER_W - _LAST_FULL * _CHUNK

_mesh = plsc.VectorSubcoreMesh(core_axis_name="c", subcore_axis_name="s")


@functools.partial(
    pl.kernel,
    out_type=jax.ShapeDtypeStruct((_B, _D), jnp.float32),
    mesh=_mesh,
    scratch_types=[
        pltpu.VMEM((_NCHUNK, _CHUNK), jnp.int32),
        [pltpu.VMEM((_CHUNK, _D), jnp.float32) for _ in range(_NBUF)],
        [pltpu.SemaphoreType.DMA for _ in range(_NBUF)],
        [pltpu.SemaphoreType.DMA for _ in range(_NBUF)],
    ],
)
def _sc_gather(x_hbm, idx_hbm, out_hbm, idx_v, bufs, gsems, ssems):
    wid = lax.axis_index("s") * 2 + lax.axis_index("c")
    base = wid * _PER_W
    pltpu.sync_copy(idx_hbm.at[wid], idx_v)

    def fire_gather(c, b):
        pltpu.async_copy(x_hbm.at[idx_v.at[c]], bufs[b], gsems[b])

    def fire_scatter(c, b):
        pltpu.async_copy(
            bufs[b], out_hbm.at[pl.ds(base + c * _CHUNK, _CHUNK)], ssems[b]
        )

    def drain(sem, b):
        pltpu.make_async_copy(x_hbm.at[pl.ds(0, _CHUNK)], bufs[b], sem).wait()

    @pl.when(wid < _NW - 1)
    def _full_worker():
        for b in range(_NBUF):
            fire_gather(b, b)

        def body(i, carry):
            for b in range(_NBUF):
                drain(gsems[b], b)
                fire_scatter(i * _NBUF + b, b)
            for b in range(_NBUF):
                drain(ssems[b], b)
                fire_gather((i + 1) * _NBUF + b, b)
            return carry

        lax.fori_loop(0, _NGRP - 1, body, 0)

        for b in range(_NBUF):
            drain(gsems[b], b)
            fire_scatter((_NGRP - 1) * _NBUF + b, b)
        for b in range(_NBUF):
            drain(ssems[b], b)

    @pl.when(wid == _NW - 1)
    def _tail_worker():
        def body(c, carry):
            pltpu.async_copy(x_hbm.at[idx_v.at[c]], bufs[0], gsems[0]).wait()
            pltpu.sync_copy(bufs[0], out_hbm.at[pl.ds(base + c * _CHUNK, _CHUNK)])
            return carry

        lax.fori_loop(0, _LAST_FULL, body, 0)
        pltpu.async_copy(x_hbm.at[idx_v.at[_LAST_FULL]], bufs[0], gsems[0]).wait()
        pltpu.sync_copy(
            bufs[0].at[pl.ds(0, _LAST_REM)],
            out_hbm.at[pl.ds(base + _LAST_FULL * _CHUNK, _LAST_REM)],
        )


def kernel(x, upsamples):
    n = upsamples.shape[0]
    idx = upsamples[:, 0].astype(jnp.int32)
    idx = jnp.concatenate([idx, jnp.zeros((_B_PAD - n,), jnp.int32)])
    idx = idx.reshape(_NW, _NCHUNK, _CHUNK)
    out = _sc_gather(x, idx)
    return out


# speedup vs baseline: 4.7132x; 1.0507x over previous
import functools

import jax
import jax.numpy as jnp
from jax import lax
from jax.experimental import pallas as pl
from jax.experimental.pallas import tpu as pltpu
from jax.experimental.pallas import tpu_sc as plsc

_D = 128
_NW = 32
_CHUNK = 64
_NBUF = 10
_NGRP = 5
_NCHUNK = _NBUF * _NGRP
_PER_W = _CHUNK * _NCHUNK
_B = 100000
_B_PAD = _PER_W * _NW
_LAST_FULL = (_B - 31 * _PER_W) // _CHUNK
_LAST_REM = _B - 31 * _PER_W - _LAST_FULL * _CHUNK

_mesh = plsc.VectorSubcoreMesh(core_axis_name="c", subcore_axis_name="s")


@functools.partial(
    pl.kernel,
    out_type=jax.ShapeDtypeStruct((_B, _D), jnp.float32),
    mesh=_mesh,
    scratch_types=[
        pltpu.VMEM((_NCHUNK, _CHUNK), jnp.int32),
        [pltpu.VMEM((_CHUNK, _D), jnp.float32) for _ in range(_NBUF)],
        [pltpu.SemaphoreType.DMA for _ in range(_NBUF)],
        [pltpu.SemaphoreType.DMA for _ in range(_NBUF)],
    ],
)
def _sc_gather(x_hbm, idx_hbm, out_hbm, idx_v, bufs, gsems, ssems):
    wid = lax.axis_index("s") * 2 + lax.axis_index("c")
    base = wid * _PER_W
    pltpu.sync_copy(idx_hbm.at[wid], idx_v)

    def fire_gather(c, b):
        pltpu.async_copy(x_hbm.at[idx_v.at[c]], bufs[b], gsems[b])

    def fire_scatter(c, b):
        pltpu.async_copy(
            bufs[b], out_hbm.at[pl.ds(base + c * _CHUNK, _CHUNK)], ssems[b]
        )

    def drain(sem, b):
        pltpu.make_async_copy(x_hbm.at[pl.ds(0, _CHUNK)], bufs[b], sem).wait()

    @pl.when(wid < _NW - 1)
    def _full_worker():
        for b in range(_NBUF):
            fire_gather(b, b)

        _H = _NBUF // 2

        def body(i, carry):
            for b in range(_NBUF):
                drain(gsems[b], b)
                fire_scatter(i * _NBUF + b, b)
                if b >= _H:
                    bb = b - _H
                    drain(ssems[bb], bb)
                    fire_gather((i + 1) * _NBUF + bb, bb)
            for bb in range(_H, _NBUF):
                drain(ssems[bb], bb)
                fire_gather((i + 1) * _NBUF + bb, bb)
            return carry

        lax.fori_loop(0, _NGRP - 1, body, 0)

        for b in range(_NBUF):
            drain(gsems[b], b)
            fire_scatter((_NGRP - 1) * _NBUF + b, b)
        for b in range(_NBUF):
            drain(ssems[b], b)

    @pl.when(wid == _NW - 1)
    def _tail_worker():
        def body(c, carry):
            pltpu.async_copy(x_hbm.at[idx_v.at[c]], bufs[0], gsems[0]).wait()
            pltpu.sync_copy(bufs[0], out_hbm.at[pl.ds(base + c * _CHUNK, _CHUNK)])
            return carry

        lax.fori_loop(0, _LAST_FULL, body, 0)
        if _LAST_REM:
            pltpu.async_copy(x_hbm.at[idx_v.at[_LAST_FULL]], bufs[0], gsems[0]).wait()
            pltpu.sync_copy(
                bufs[0].at[pl.ds(0, _LAST_REM)],
                out_hbm.at[pl.ds(base + _LAST_FULL * _CHUNK, _LAST_REM)],
            )


def kernel(x, upsamples):
    n = upsamples.shape[0]
    idx = upsamples[:, 0].astype(jnp.int32)
    idx = jnp.concatenate([idx, jnp.zeros((_B_PAD - n,), jnp.int32)])
    idx = idx.reshape(_NW, _NCHUNK, _CHUNK)
    out = _sc_gather(x, idx)
    return out
